# baseline (speedup 1.0000x reference)
import jax
import jax.numpy as jnp
from jax.experimental import pallas as pl
from jax.experimental.pallas import tpu as pltpu

_INDEX = 2


_BLOCK_ROWS = 256


def _copy_kernel(x_vmem, o_vmem):
    o_vmem[...] = x_vmem[0]


def kernel(x):
    _, rows, cols = x.shape
    grid = rows // _BLOCK_ROWS
    return pl.pallas_call(
        _copy_kernel,
        grid=(grid,),
        in_specs=[
            pl.BlockSpec((1, _BLOCK_ROWS, cols), lambda i: (_INDEX, i, 0))
        ],
        out_specs=pl.BlockSpec((_BLOCK_ROWS, cols), lambda i: (i, 0)),
        out_shape=jax.ShapeDtypeStruct(x.shape[1:], x.dtype),
        compiler_params=pltpu.CompilerParams(
            dimension_semantics=("parallel",),
        ),
    )(x)


# speedup vs baseline: 1.0167x; 1.0167x over previous
import jax
import jax.numpy as jnp
from jax.experimental import pallas as pl
from jax.experimental.pallas import tpu as pltpu

_INDEX = 2

_ROWS_PER_CHUNK = 256
_NSLOTS = 8
_DEPTH = 4


def _copy_kernel(x_hbm, o_hbm, buf, sem_in, sem_out):
    rows, cols = o_hbm.shape
    nsteps = rows // _ROWS_PER_CHUNK

    def in_copy(i):
        return pltpu.make_async_copy(
            x_hbm.at[_INDEX, pl.ds(i * _ROWS_PER_CHUNK, _ROWS_PER_CHUNK), :],
            buf.at[i % _NSLOTS],
            sem_in.at[i % _NSLOTS],
        )

    def out_copy(i):
        return pltpu.make_async_copy(
            buf.at[i % _NSLOTS],
            o_hbm.at[pl.ds(i * _ROWS_PER_CHUNK, _ROWS_PER_CHUNK), :],
            sem_out.at[i % _NSLOTS],
        )

    for i in range(min(_DEPTH, nsteps)):
        in_copy(i).start()
    for i in range(nsteps):
        in_copy(i).wait()
        out_copy(i).start()
        j = i + _DEPTH
        if j < nsteps:
            if j - _NSLOTS >= 0:
                out_copy(j - _NSLOTS).wait()
            in_copy(j).start()
    for i in range(max(0, nsteps - _NSLOTS), nsteps):
        out_copy(i).wait()


def kernel(x):
    _, rows, cols = x.shape
    return pl.pallas_call(
        _copy_kernel,
        out_shape=jax.ShapeDtypeStruct(x.shape[1:], x.dtype),
        in_specs=[pl.BlockSpec(memory_space=pltpu.MemorySpace.HBM)],
        out_specs=pl.BlockSpec(memory_space=pltpu.MemorySpace.HBM),
        scratch_shapes=[
            pltpu.VMEM((_NSLOTS, _ROWS_PER_CHUNK, cols), x.dtype),
            pltpu.SemaphoreType.DMA((_NSLOTS,)),
            pltpu.SemaphoreType.DMA((_NSLOTS,)),
        ],
    )(x)


# speedup vs baseline: 1.0183x; 1.0015x over previous
import jax
import jax.numpy as jnp
from jax.experimental import pallas as pl
from jax.experimental.pallas import tpu as pltpu

_INDEX = 2

_ROWS_PER_CHUNK = 256
_NSLOTS = 8
_DEPTH = 4


def _copy_kernel(x_hbm, o_hbm, buf, sem_in, sem_out):
    rows, cols = o_hbm.shape
    nsteps = rows // _ROWS_PER_CHUNK

    def in_copy(i):
        return pltpu.make_async_copy(
            x_hbm.at[_INDEX, pl.ds(i * _ROWS_PER_CHUNK, _ROWS_PER_CHUNK), :],
            buf.at[i % _NSLOTS],
            sem_in.at[i % _NSLOTS],
        )

    def out_copy(i):
        return pltpu.make_async_copy(
            buf.at[i % _NSLOTS],
            o_hbm.at[pl.ds(i * _ROWS_PER_CHUNK, _ROWS_PER_CHUNK), :],
            sem_out.at[i % _NSLOTS],
        )

    for i in range(min(_DEPTH, nsteps)):
        in_copy(i).start()
    for i in range(nsteps):
        in_copy(i).wait()
        out_copy(i).start()
        j = i + _DEPTH
        if j < nsteps:
            if j - _NSLOTS >= 0:
                out_copy(j - _NSLOTS).wait()
            in_copy(j).start()
    for i in range(max(0, nsteps - _NSLOTS), nsteps):
        out_copy(i).wait()


def kernel(x):
    _, rows, cols = x.shape
    return pl.pallas_call(
        _copy_kernel,
        out_shape=jax.ShapeDtypeStruct(x.shape[1:], x.dtype),
        in_specs=[pl.BlockSpec(memory_space=pltpu.MemorySpace.HBM)],
        out_specs=pl.BlockSpec(memory_space=pltpu.MemorySpace.HBM),
        scratch_shapes=[
            pltpu.VMEM((_NSLOTS, _ROWS_PER_CHUNK, cols), x.dtype),
            pltpu.SemaphoreType.DMA((_NSLOTS,)),
            pltpu.SemaphoreType.DMA((_NSLOTS,)),
        ],
        compiler_params=pltpu.CompilerParams(skip_device_barrier=True),
    )(x)


# speedup vs baseline: 1.0191x; 1.0008x over previous
import jax
import jax.numpy as jnp
from jax.experimental import pallas as pl
from jax.experimental.pallas import tpu as pltpu

_INDEX = 2

_ROWS_PER_CHUNK = 512
_NSLOTS = 6
_DEPTH = 3


def _copy_kernel(x_hbm, o_hbm, buf, sem_in, sem_out):
    rows, cols = o_hbm.shape
    nsteps = rows // _ROWS_PER_CHUNK

    def in_copy(i):
        return pltpu.make_async_copy(
            x_hbm.at[_INDEX, pl.ds(i * _ROWS_PER_CHUNK, _ROWS_PER_CHUNK), :],
            buf.at[i % _NSLOTS],
            sem_in.at[i % _NSLOTS],
        )

    def out_copy(i):
        return pltpu.make_async_copy(
            buf.at[i % _NSLOTS],
            o_hbm.at[pl.ds(i * _ROWS_PER_CHUNK, _ROWS_PER_CHUNK), :],
            sem_out.at[i % _NSLOTS],
        )

    for i in range(min(_DEPTH, nsteps)):
        in_copy(i).start()
    for i in range(nsteps):
        in_copy(i).wait()
        out_copy(i).start()
        j = i + _DEPTH
        if j < nsteps:
            if j - _NSLOTS >= 0:
                out_copy(j - _NSLOTS).wait()
            in_copy(j).start()
    for i in range(max(0, nsteps - _NSLOTS), nsteps):
        out_copy(i).wait()


def kernel(x):
    _, rows, cols = x.shape
    return pl.pallas_call(
        _copy_kernel,
        out_shape=jax.ShapeDtypeStruct(x.shape[1:], x.dtype),
        in_specs=[pl.BlockSpec(memory_space=pltpu.MemorySpace.HBM)],
        out_specs=pl.BlockSpec(memory_space=pltpu.MemorySpace.HBM),
        scratch_shapes=[
            pltpu.VMEM((_NSLOTS, _ROWS_PER_CHUNK, cols), x.dtype),
            pltpu.SemaphoreType.DMA((_NSLOTS,)),
            pltpu.SemaphoreType.DMA((_NSLOTS,)),
        ],
        compiler_params=pltpu.CompilerParams(skip_device_barrier=True),
    )(x)


# speedup vs baseline: 1.0202x; 1.0011x over previous
import jax
import jax.numpy as jnp
from jax.experimental import pallas as pl
from jax.experimental.pallas import tpu as pltpu

_INDEX = 2

_ROWS_PER_CHUNK = 512
_NSLOTS = 4
_DEPTH = 2


def _copy_kernel(x_hbm, o_hbm, buf, sem_in, sem_out):
    rows, cols = o_hbm.shape
    nsteps = rows // _ROWS_PER_CHUNK

    def in_copy(i):
        return pltpu.make_async_copy(
            x_hbm.at[_INDEX, pl.ds(i * _ROWS_PER_CHUNK, _ROWS_PER_CHUNK), :],
            buf.at[i % _NSLOTS],
            sem_in.at[i % _NSLOTS],
        )

    def out_copy(i):
        return pltpu.make_async_copy(
            buf.at[i % _NSLOTS],
            o_hbm.at[pl.ds(i * _ROWS_PER_CHUNK, _ROWS_PER_CHUNK), :],
            sem_out.at[i % _NSLOTS],
        )

    for i in range(min(_DEPTH, nsteps)):
        in_copy(i).start()
    for i in range(nsteps):
        in_copy(i).wait()
        out_copy(i).start()
        j = i + _DEPTH
        if j < nsteps:
            if j - _NSLOTS >= 0:
                out_copy(j - _NSLOTS).wait()
            in_copy(j).start()
    for i in range(max(0, nsteps - _NSLOTS), nsteps):
        out_copy(i).wait()


def kernel(x):
    _, rows, cols = x.shape
    return pl.pallas_call(
        _copy_kernel,
        out_shape=jax.ShapeDtypeStruct(x.shape[1:], x.dtype),
        in_specs=[pl.BlockSpec(memory_space=pltpu.MemorySpace.HBM)],
        out_specs=pl.BlockSpec(memory_space=pltpu.MemorySpace.HBM),
        scratch_shapes=[
            pltpu.VMEM((_NSLOTS, _ROWS_PER_CHUNK, cols), x.dtype),
            pltpu.SemaphoreType.DMA((_NSLOTS,)),
            pltpu.SemaphoreType.DMA((_NSLOTS,)),
        ],
        compiler_params=pltpu.CompilerParams(skip_device_barrier=True),
    )(x)
